# baseline (speedup 1.0000x reference)
import functools

import jax
import jax.numpy as jnp
from jax import lax
from jax.experimental import pallas as pl
from jax.experimental.pallas import tpu as pltpu
from jax.experimental.pallas import tpu_sc as plsc

NC = 2
NS = 16
CL = 20


def kernel(feat, spatial_emb, temporal_emb):
    B, L, O, D = feat.shape
    NK = D // 16
    mesh = plsc.VectorSubcoreMesh(core_axis_name="c", subcore_axis_name="s")

    @functools.partial(
        pl.kernel,
        mesh=mesh,
        out_type=jax.ShapeDtypeStruct((B, L, O, D), jnp.float32),
        scratch_types=[
            pltpu.VMEM((CL, O, D), jnp.float32),
            pltpu.VMEM((L, D), jnp.float32),
            pltpu.VMEM((O, D), jnp.float32),
            pltpu.SemaphoreType.DMA,
        ],
    )
    def k(t_hbm, s_hbm, f_hbm, o_hbm, buf, t_v, s_v, sem):
        wid = lax.axis_index("s") * NC + lax.axis_index("c")
        pltpu.sync_copy(t_hbm, t_v)
        pltpu.sync_copy(s_hbm, s_v)
        ncl = L // CL

        def chunk_body(ci, carry):
            b = (B // (NC * NS)) * wid + lax.div(ci, ncl)
            l0 = lax.rem(ci, ncl) * CL
            pltpu.async_copy(f_hbm.at[b, pl.ds(l0, CL)], buf, sem).wait()

            def l_body(lg, c2):
                for u in range(4):
                    l = lg * 4 + u
                    tv = [t_v[l0 + l, pl.ds(kk * 16, 16)] for kk in range(NK)]
                    for o in range(O):
                        for kk in range(NK):
                            sl = pl.ds(kk * 16, 16)
                            buf[l, o, sl] = buf[l, o, sl] + (tv[kk] + s_v[o, sl])
                return c2

            lax.fori_loop(0, CL // 4, l_body, 0)
            pltpu.async_copy(buf, o_hbm.at[b, pl.ds(l0, CL)], sem).wait()
            return carry

        lax.fori_loop(0, (B // (NC * NS)) * ncl, chunk_body, 0)

    return k(temporal_emb, spatial_emb, feat)


# speedup vs baseline: 1.3325x; 1.3325x over previous
import functools

import jax
import jax.numpy as jnp
from jax import lax
from jax.experimental import pallas as pl
from jax.experimental.pallas import tpu as pltpu
from jax.experimental.pallas import tpu_sc as plsc

NC = 2
NS = 16
CL = 10


def kernel(feat, spatial_emb, temporal_emb):
    B, L, O, D = feat.shape
    NK = D // 16
    mesh = plsc.VectorSubcoreMesh(core_axis_name="c", subcore_axis_name="s")

    @functools.partial(
        pl.kernel,
        mesh=mesh,
        out_type=jax.ShapeDtypeStruct((B, L, O, D), jnp.float32),
        scratch_types=[
            pltpu.VMEM((CL, O, D), jnp.float32),
            pltpu.VMEM((CL, O, D), jnp.float32),
            pltpu.VMEM((L, D), jnp.float32),
            pltpu.VMEM((O, D), jnp.float32),
            pltpu.SemaphoreType.DMA,
        ],
    )
    def k(t_hbm, s_hbm, f_hbm, o_hbm, buf, obuf, t_v, s_v, sem):
        wid = lax.axis_index("s") * NC + lax.axis_index("c")
        pltpu.sync_copy(t_hbm, t_v)
        pltpu.sync_copy(s_hbm, s_v)
        ncl = L // CL

        def chunk_body(ci, carry):
            b = (B // (NC * NS)) * wid + lax.div(ci, ncl)
            l0 = lax.rem(ci, ncl) * CL
            pltpu.async_copy(f_hbm.at[b, pl.ds(l0, CL)], buf, sem).wait()

            def l_body(l, c2):
                tv = [t_v[l0 + l, pl.ds(kk * 16, 16)] for kk in range(NK)]
                for o in range(O):
                    for kk in range(NK):
                        sl = pl.ds(kk * 16, 16)
                        obuf[l, o, sl] = buf[l, o, sl] + (tv[kk] + s_v[o, sl])
                return c2

            lax.fori_loop(0, CL, l_body, 0)
            pltpu.async_copy(obuf, o_hbm.at[b, pl.ds(l0, CL)], sem).wait()
            return carry

        lax.fori_loop(0, (B // (NC * NS)) * ncl, chunk_body, 0)

    return k(temporal_emb, spatial_emb, feat)


# speedup vs baseline: 1.5377x; 1.1540x over previous
import functools

import jax
import jax.numpy as jnp
from jax import lax
from jax.experimental import pallas as pl
from jax.experimental.pallas import tpu as pltpu
from jax.experimental.pallas import tpu_sc as plsc

NC = 2
NS = 16
CL = 5


def kernel(feat, spatial_emb, temporal_emb):
    B, L, O, D = feat.shape
    NK = D // 16
    BPW = B // (NC * NS)
    NCH = BPW * (L // CL)
    mesh = plsc.VectorSubcoreMesh(core_axis_name="c", subcore_axis_name="s")

    @functools.partial(
        pl.kernel,
        mesh=mesh,
        out_type=jax.ShapeDtypeStruct((B, L, O, D), jnp.float32),
        scratch_types=[
            pltpu.VMEM((CL, O, D), jnp.float32),
            pltpu.VMEM((CL, O, D), jnp.float32),
            pltpu.VMEM((CL, O, D), jnp.float32),
            pltpu.VMEM((CL, O, D), jnp.float32),
            pltpu.VMEM((L, D), jnp.float32),
            pltpu.VMEM((O, D), jnp.float32),
            pltpu.SemaphoreType.DMA,
            pltpu.SemaphoreType.DMA,
            pltpu.SemaphoreType.DMA,
            pltpu.SemaphoreType.DMA,
        ],
    )
    def k(t_hbm, s_hbm, f_hbm, o_hbm, i0, i1, o0, o1, t_v, s_v,
          is0, is1, os0, os1):
        wid = lax.axis_index("s") * NC + lax.axis_index("c")
        pltpu.sync_copy(t_hbm, t_v)
        pltpu.sync_copy(s_hbm, s_v)
        ncl = L // CL
        b0 = BPW * wid

        def src(c):
            return f_hbm.at[b0 + lax.div(c, ncl),
                            pl.ds(lax.rem(c, ncl) * CL, CL)]

        def dst(c):
            return o_hbm.at[b0 + lax.div(c, ncl),
                            pl.ds(lax.rem(c, ncl) * CL, CL)]

        pltpu.make_async_copy(src(0), i0, is0).start()
        pltpu.make_async_copy(src(1), i1, is1).start()

        def compute(ibuf, obuf, l0):
            def l_body(l, c2):
                tv = [t_v[l0 + l, pl.ds(kk * 16, 16)] for kk in range(NK)]
                for o in range(O):
                    for kk in range(NK):
                        sl = pl.ds(kk * 16, 16)
                        obuf[l, o, sl] = ibuf[l, o, sl] + (tv[kk] + s_v[o, sl])
                return c2
            lax.fori_loop(0, CL, l_body, 0)

        def round_body(g, carry):
            for ibuf, obuf, isem, osem, par in ((i0, o0, is0, os0, 0),
                                                (i1, o1, is1, os1, 1)):
                c = 2 * g + par
                pltpu.make_async_copy(src(c), ibuf, isem).wait()

                @pl.when(c >= 2)
                def _():
                    pltpu.make_async_copy(obuf, dst(c - 2), osem).wait()

                compute(ibuf, obuf, lax.rem(c, ncl) * CL)
                pltpu.make_async_copy(obuf, dst(c), osem).start()

                @pl.when(c + 2 < NCH)
                def _():
                    pltpu.make_async_copy(src(c + 2), ibuf, isem).start()
            return carry

        lax.fori_loop(0, NCH // 2, round_body, 0)

        pltpu.make_async_copy(o0, dst(NCH - 2), os0).wait()
        pltpu.make_async_copy(o1, dst(NCH - 1), os1).wait()

    return k(temporal_emb, spatial_emb, feat)
